# baseline (speedup 1.0000x reference)
import functools

import jax
import jax.numpy as jnp
from jax import lax
from jax.experimental import pallas as pl
from jax.experimental.pallas import tpu as pltpu
from jax.experimental.pallas import tpu_sc as plsc

NU = 50000
NI = 50000
N = NU + NI
E = 800000
D = 64
H0 = 64
H2 = 32
R = 5
CF = 16
FH = 64

HALF = 25000
ACC_ROWS = 25088
PAD0 = HALF
CE = 400
EPT = E // 16
NCHUNK = EPT // CE
TROWS = ACC_ROWS // 16


def _tables_body(feats_ref, wg_ref, wc_ref, t_ref, c_ref):
    w = wg_ref[...]
    blocks = [w[0]]
    for r in range(1, R):
        blocks.append(blocks[-1] + w[r])
    blocks.append(wc_ref[...])
    wst = jnp.concatenate(blocks, axis=1)
    y = jnp.dot(feats_ref[...], wst, preferred_element_type=jnp.float32)
    for r in range(R):
        t_ref[r] = y[:, r * H0:(r + 1) * H0]
    c_ref[...] = y[:, R * H0:]


def _make_tables(feats, w_gcn, w_cgcn):
    nb = 2000
    grid = (N // nb,)
    return pl.pallas_call(
        _tables_body,
        grid=grid,
        in_specs=[
            pl.BlockSpec((nb, D), lambda i: (i, 0)),
            pl.BlockSpec((R, D, H0), lambda i: (0, 0, 0)),
            pl.BlockSpec((D, H0), lambda i: (0, 0)),
        ],
        out_specs=[
            pl.BlockSpec((R, nb, H0), lambda i: (0, i, 0)),
            pl.BlockSpec((nb, H0), lambda i: (i, 0)),
        ],
        out_shape=[
            jax.ShapeDtypeStruct((R, N, H0), jnp.float32),
            jax.ShapeDtypeStruct((N, H0), jnp.float32),
        ],
    )(feats, w_gcn, w_cgcn)


def _eidx_body(row_ref, col_ref, et_ref, gu_ref, gv_ref, cu_ref):
    row = row_ref[...]
    col = col_ref[...]
    et = et_ref[...]
    gu_ref[...] = et * N + (col + NU)
    gv_ref[...] = et * N + row
    cu_ref[...] = col + NU


def _edge_indices(row2d, col2d, et2d):
    grid = (1,)
    spec = pl.BlockSpec(row2d.shape, lambda i: (0, 0))
    return pl.pallas_call(
        _eidx_body,
        grid=grid,
        in_specs=[spec, spec, spec],
        out_specs=[spec, spec, spec],
        out_shape=[jax.ShapeDtypeStruct(row2d.shape, jnp.int32)] * 3,
    )(row2d, col2d, et2d)


HALVES = ((0, 208), (208, 192))


def _agg_body(with_w, *refs):
    if with_w:
        (dst_hbm, idx_hbm, w_hbm, tbl_hbm, acc_out, ed_dst, ed_idx, ed_w,
         dstloc0, dstloc1, payload, sem, sem_s0, sem_s1, acc_sh) = refs
    else:
        (dst_hbm, idx_hbm, tbl_hbm, acc_out, ed_dst, ed_idx,
         dstloc0, dstloc1, payload, sem, sem_s0, sem_s1, acc_sh) = refs
    dstlocs = (dstloc0, dstloc1)
    ssems = (sem_s0, sem_s1)

    cid = lax.axis_index("c")
    sid = lax.axis_index("s")
    ii = lax.iota(jnp.int32, 16)
    z16 = jnp.zeros((16,), jnp.float32)

    def zp(i, carry):
        r16 = jnp.full((16,), i, jnp.int32)
        for s in range(4):
            plsc.store_scatter(payload, [r16, ii + 16 * s], z16)
        return carry
    lax.fori_loop(0, CE, zp, 0)

    row0 = sid * TROWS
    for k in range(4):
        pltpu.sync_copy(payload.at[pl.ds(0, 392)],
                        acc_sh.at[pl.ds(row0 + 392 * k, 392)])
    plsc.subcore_barrier()

    ebase = sid * EPT
    lo = cid * HALF

    def load_chunk(c, bb):
        e0 = ebase + c * CE
        pltpu.sync_copy(dst_hbm.at[pl.ds(e0, CE)], ed_dst)
        pltpu.sync_copy(idx_hbm.at[pl.ds(e0, CE)], ed_idx.at[bb])
        if with_w:
            pltpu.sync_copy(w_hbm.at[pl.ds(e0, CE)], ed_w.at[bb])
        for v in range(CE // 16):
            d = ed_dst[pl.ds(16 * v, 16)]
            local = d - lo
            inr = (local >= 0) & (local < HALF)
            padr = PAD0 + ((ii + 16 * v) & 63)
            loc2 = jnp.where(inr, local, padr)
            if 16 * v < HALVES[0][1]:
                dstloc0[bb, pl.ds(16 * v, 16)] = loc2
            else:
                dstloc1[bb, pl.ds(16 * v - HALVES[0][1], 16)] = loc2

    def gather(bb, h):
        off, ln = HALVES[h]
        return pltpu.make_async_copy(
            tbl_hbm.at[ed_idx.at[bb, pl.ds(off, ln)]],
            payload.at[pl.ds(off, ln)], sem)

    def issue_gather(bb, h):
        off, ln = HALVES[h]
        pltpu.async_copy(
            tbl_hbm.at[ed_idx.at[bb, pl.ds(off, ln)]],
            payload.at[pl.ds(off, ln)], sem)

    def wmul_half(bb, h):
        off, ln = HALVES[h]

        def wmul(g, c2):
            for u in range(8):
                j = g * 8 + u
                wv = plsc.load_gather(
                    ed_w, [jnp.full((16,), bb, jnp.int32),
                           jnp.full((16,), off + j, jnp.int32)])
                for s in range(4):
                    sl = (off + j, pl.ds(16 * s, 16))
                    payload[sl] = payload[sl] * wv
            return c2
        lax.fori_loop(0, ln // 8, wmul, 0)

    def scatter_half(bb, h):
        off, ln = HALVES[h]
        pltpu.async_copy(payload.at[pl.ds(off, ln)],
                         acc_sh.at[dstlocs[h].at[bb]], ssems[h], add=True)

    def wait_scatter(bb, h):
        off, ln = HALVES[h]
        pltpu.make_async_copy(payload.at[pl.ds(off, ln)],
                              acc_sh.at[dstlocs[h].at[bb]], ssems[h]).wait()

    load_chunk(0, 0)
    issue_gather(0, 0)

    def chunk(c, carry):
        bb = c & 1
        nbb = 1 - bb

        @pl.when(c > 0)
        def _():
            wait_scatter(nbb, 1)

        @pl.when(c + 1 < NCHUNK)
        def _():
            load_chunk(c + 1, nbb)
        gather(bb, 0).wait()
        issue_gather(bb, 1)
        if with_w:
            wmul_half(bb, 0)
        scatter_half(bb, 0)
        gather(bb, 1).wait()
        wait_scatter(bb, 0)

        @pl.when(c + 1 < NCHUNK)
        def _():
            issue_gather(nbb, 0)
        if with_w:
            wmul_half(bb, 1)
        scatter_half(bb, 1)
        return carry
    lax.fori_loop(0, NCHUNK, chunk, 0)
    wait_scatter((NCHUNK - 1) & 1, 1)
    plsc.subcore_barrier()

    pltpu.sync_copy(acc_sh.at[pl.ds(row0, TROWS)],
                    acc_out.at[pl.ds(cid * ACC_ROWS + row0, TROWS)])


_SC_PARAMS = pltpu.CompilerParams(
    needs_layout_passes=False, use_tc_tiling_on_sc=False)


def _make_agg(with_w):
    mesh = plsc.VectorSubcoreMesh(core_axis_name="c", subcore_axis_name="s")
    out_type = [jax.ShapeDtypeStruct((2 * ACC_ROWS, H0), jnp.float32)]
    scratch = [
        pltpu.VMEM((CE,), jnp.int32),
        pltpu.VMEM((2, CE), jnp.int32),
    ]
    if with_w:
        scratch.append(pltpu.VMEM((2, CE), jnp.float32))
    scratch += [
        pltpu.VMEM((2, HALVES[0][1]), jnp.int32),
        pltpu.VMEM((2, HALVES[1][1]), jnp.int32),
        pltpu.VMEM((CE, H0), jnp.float32),
        pltpu.SemaphoreType.DMA,
        pltpu.SemaphoreType.DMA,
        pltpu.SemaphoreType.DMA,
    ]
    scratch.append(pltpu.VMEM_SHARED((ACC_ROWS, H0), jnp.float32))
    return functools.partial(
        pl.kernel, mesh=mesh, out_type=out_type, scratch_types=scratch,
        compiler_params=_SC_PARAMS,
    )(functools.partial(_agg_body, with_w))


DEG_ROWS = 50176


def _deg_body(dst2_hbm, deg_out, ed_dst, degloc):
    cid = lax.axis_index("c")
    sid = lax.axis_index("s")
    ii = lax.iota(jnp.int32, 16)
    z16 = jnp.zeros((16,), jnp.float32)
    ones16 = jnp.ones((16,), jnp.float32)

    def zd(i, carry):
        plsc.store_scatter(degloc, [i * 16 + ii], z16)
        return carry
    lax.fori_loop(0, DEG_ROWS // 16, zd, 0)

    ebase = cid * E + sid * EPT

    def chunk(i, carry):
        pltpu.sync_copy(dst2_hbm.at[pl.ds(ebase + i * CE, CE)], ed_dst)
        for v in range(CE // 16):
            d = ed_dst[pl.ds(16 * v, 16)]
            plsc.addupdate_scatter(degloc, [d], ones16)
        return carry
    lax.fori_loop(0, NCHUNK, chunk, 0)
    pltpu.sync_copy(degloc, deg_out.at[cid * 16 + sid])


def _make_deg():
    mesh = plsc.VectorSubcoreMesh(core_axis_name="c", subcore_axis_name="s")
    return functools.partial(
        pl.kernel, mesh=mesh,
        out_type=[jax.ShapeDtypeStruct((32, DEG_ROWS), jnp.float32)],
        scratch_types=[
            pltpu.VMEM((CE,), jnp.int32),
            pltpu.VMEM((DEG_ROWS,), jnp.float32),
        ],
        compiler_params=_SC_PARAMS,
    )(_deg_body)


def _combine_body(accg_ref, accc_ref, deg_ref, ctx_ref, wf_ref, bf_ref,
                  w1_ref, w2_ref, out_ref):
    dg = jnp.maximum(jnp.sum(deg_ref[...], axis=1), 1.0)
    recip = (1.0 / dg)[:, None]
    g = jax.nn.relu(accg_ref[...] * recip)
    c = jax.nn.relu(accc_ref[...] * recip)
    h = jax.nn.relu(
        jnp.dot(ctx_ref[...], wf_ref[...],
                preferred_element_type=jnp.float32) + bf_ref[...])
    w1 = w1_ref[...]
    w2 = w2_ref[...]
    out = jnp.dot(g, w1[:H0] + w2[:H0], preferred_element_type=jnp.float32)
    out += jnp.dot(h, w1[H0:], preferred_element_type=jnp.float32)
    out += jnp.dot(c, w2[H0:], preferred_element_type=jnp.float32)
    out_ref[...] = out


def _combine(accg, accc, deg, ctx, wf, bf, w1, w2):
    nb = 1000
    nblk = NU // nb
    hb = HALF // nb
    return pl.pallas_call(
        _combine_body,
        grid=(nblk,),
        in_specs=[
            pl.BlockSpec((nb, H0), lambda i: (i, 0)),
            pl.BlockSpec((nb, H0), lambda i: (i, 0)),
            pl.BlockSpec((nb, 16), lambda i: (i, 0)),
            pl.BlockSpec((nb, CF), lambda i: (i, 0)),
            pl.BlockSpec((CF, FH), lambda i: (0, 0)),
            pl.BlockSpec((1, FH), lambda i: (0, 0)),
            pl.BlockSpec((H0 + FH, H2), lambda i: (0, 0)),
            pl.BlockSpec((2 * H0, H2), lambda i: (0, 0)),
        ],
        out_specs=pl.BlockSpec((nb, H2), lambda i: (i, 0)),
        out_shape=jax.ShapeDtypeStruct((NU, H2), jnp.float32),
    )(accg, accc, deg, ctx, wf, bf, w1, w2)


def _halves(acc):
    return jnp.concatenate([acc[:HALF], acc[ACC_ROWS:ACC_ROWS + HALF]], axis=0)


def kernel(u_features, v_features, edge_index, edge_type, edge_ctx_weight,
           u_context, v_context, W_gcn, W_cgcn, W_fu, b_fu, W_fv, b_fv,
           W1_u, W1_v, W2_u, W2_v):
    row = edge_index[0].astype(jnp.int32)
    col = edge_index[1].astype(jnp.int32)
    et = edge_type.astype(jnp.int32)
    w = edge_ctx_weight.astype(jnp.float32)

    feats = jnp.concatenate([u_features, v_features], axis=0)
    t_tbl, c_tbl = _make_tables(feats, W_gcn, W_cgcn)
    t_flat = t_tbl.reshape(R * N, H0)

    e2 = (E // 128, 128)
    gu2, gv2, cu2 = _edge_indices(row.reshape(e2), col.reshape(e2),
                                  et.reshape(e2))
    gu = gu2.reshape(E)
    gv = gv2.reshape(E)
    cu = cu2.reshape(E)

    agg_gcn = _make_agg(False)
    agg_ctx = _make_agg(True)

    (accu_g,) = agg_gcn(row, gu, t_flat)
    (accu_c,) = agg_ctx(row, cu, w, c_tbl)
    (accv_g,) = agg_gcn(col, gv, t_flat)
    (accv_c,) = agg_ctx(col, row, w, c_tbl)
    (deg2,) = _make_deg()(jnp.concatenate([row, col]))
    degu = deg2[:16, :NU].T
    degv = deg2[16:, :NU].T

    bfu = b_fu.reshape(1, FH)
    bfv = b_fv.reshape(1, FH)
    out_u = _combine(_halves(accu_g), _halves(accu_c), degu,
                     u_context, W_fu, bfu, W1_u, W2_u)
    out_v = _combine(_halves(accv_g), _halves(accv_c), degv,
                     v_context, W_fv, bfv, W1_v, W2_v)
    return jnp.concatenate([out_u, out_v], axis=0)


# speedup vs baseline: 1.1914x; 1.1914x over previous
import functools

import jax
import jax.numpy as jnp
from jax import lax
from jax.experimental import pallas as pl
from jax.experimental.pallas import tpu as pltpu
from jax.experimental.pallas import tpu_sc as plsc

NU = 50000
NI = 50000
N = NU + NI
E = 800000
D = 64
H0 = 64
H2 = 32
R = 5
CF = 16
FH = 64

HALF = 25000
ACC_ROWS = 25088
PAD0 = HALF
CE = 400
EPT = E // 16
NCHUNK = EPT // CE
TROWS = ACC_ROWS // 16


def _tables_body(feats_ref, wg_ref, wc_ref, t_ref, c_ref):
    w = wg_ref[...]
    blocks = [w[0]]
    for r in range(1, R):
        blocks.append(blocks[-1] + w[r])
    blocks.append(wc_ref[...])
    wst = jnp.concatenate(blocks, axis=1)
    y = jnp.dot(feats_ref[...], wst, preferred_element_type=jnp.float32)
    for r in range(R):
        t_ref[r] = y[:, r * H0:(r + 1) * H0]
    c_ref[...] = y[:, R * H0:]


def _make_tables(feats, w_gcn, w_cgcn):
    nb = 2000
    grid = (N // nb,)
    return pl.pallas_call(
        _tables_body,
        grid=grid,
        in_specs=[
            pl.BlockSpec((nb, D), lambda i: (i, 0)),
            pl.BlockSpec((R, D, H0), lambda i: (0, 0, 0)),
            pl.BlockSpec((D, H0), lambda i: (0, 0)),
        ],
        out_specs=[
            pl.BlockSpec((R, nb, H0), lambda i: (0, i, 0)),
            pl.BlockSpec((nb, H0), lambda i: (i, 0)),
        ],
        out_shape=[
            jax.ShapeDtypeStruct((R, N, H0), jnp.float32),
            jax.ShapeDtypeStruct((N, H0), jnp.float32),
        ],
    )(feats, w_gcn, w_cgcn)


def _eidx_body(row_ref, col_ref, et_ref, gu_ref, gv_ref, cu_ref):
    row = row_ref[...]
    col = col_ref[...]
    et = et_ref[...]
    gu_ref[...] = et * N + (col + NU)
    gv_ref[...] = et * N + row
    cu_ref[...] = col + NU


def _edge_indices(row2d, col2d, et2d):
    grid = (1,)
    spec = pl.BlockSpec(row2d.shape, lambda i: (0, 0))
    return pl.pallas_call(
        _eidx_body,
        grid=grid,
        in_specs=[spec, spec, spec],
        out_specs=[spec, spec, spec],
        out_shape=[jax.ShapeDtypeStruct(row2d.shape, jnp.int32)] * 3,
    )(row2d, col2d, et2d)


REGC = EPT + CE
REG = 32 * REGC
STG = CE + 16


def _part_body(dst2_hbm, gidx2_hbm, cidx2_hbm, w_hbm,
               dstp, gidxp, cidxp, wp, cnt_out,
               b_dst, b_gidx, b_cidx, b_w, *stg):
    cid = lax.axis_index("c")
    sid = lax.axis_index("s")
    ii = lax.iota(jnp.int32, 16)
    S = (stg[0:4], stg[4:8])
    outs = (dstp, gidxp, cidxp, wp)

    def flush(h, nf):
        base = (cid * 32 + h * 16 + sid) * REGC + nf * CE
        for a in range(4):
            pltpu.sync_copy(S[h][a].at[pl.ds(0, CE)],
                            outs[a].at[pl.ds(base, CE)])
        for a in range(4):
            t16 = S[h][a][pl.ds(CE, 16)]
            S[h][a][pl.ds(0, 16)] = t16

    def chunk(i, carry):
        off0, nf0, off1, nf1 = carry
        e0 = cid * E + sid * EPT + i * CE
        ew = sid * EPT + i * CE
        pltpu.sync_copy(dst2_hbm.at[pl.ds(e0, CE)], b_dst)
        pltpu.sync_copy(gidx2_hbm.at[pl.ds(e0, CE)], b_gidx)
        pltpu.sync_copy(cidx2_hbm.at[pl.ds(e0, CE)], b_cidx)
        pltpu.sync_copy(w_hbm.at[pl.ds(ew, CE)], b_w)
        for v in range(CE // 16):
            sl = pl.ds(16 * v, 16)
            d = b_dst[sl]
            vals = (d, b_gidx[sl], b_cidx[sl], b_w[sl])
            m0 = d < HALF
            offs = [off0, off1]
            nfs = [nf0, nf1]
            for h in range(2):
                m = m0 if h == 0 else jnp.logical_not(m0)
                off = offs[h]
                for a in range(4):
                    plsc.store_compressed(S[h][a].at[pl.ds(off, 16)],
                                          vals[a], mask=m)
                off = off + jnp.sum(m.astype(jnp.int32))
                do = off >= CE

                @pl.when(do)
                def _(h=h, nf=nfs[h]):
                    flush(h, nf)
                offs[h] = jnp.where(do, off - CE, off)
                nfs[h] = nfs[h] + do.astype(jnp.int32)
            off0, off1 = offs
            nf0, nf1 = nfs
        return off0, nf0, off1, nf1
    off0, nf0, off1, nf1 = lax.fori_loop(
        0, NCHUNK, chunk, (jnp.int32(0), jnp.int32(0),
                           jnp.int32(0), jnp.int32(0)))

    dpad = jnp.full((16,), -1, jnp.int32)
    spread = ii & 63
    wpad = jnp.zeros((16,), jnp.float32)
    pads = (dpad, spread, spread, wpad)
    for h, off, nf in ((0, off0, nf0), (1, off1, nf1)):
        for k in range(CE // 16):
            @pl.when(off + 16 * k < CE)
            def _(h=h, k=k, off=off):
                for a in range(4):
                    S[h][a][pl.ds(off + 16 * k, 16)] = pads[a]
        flush(h, nf)
        S[0][0][pl.ds(0, 16)] = jnp.full((16,), nf + 1, jnp.int32)
        pltpu.sync_copy(S[0][0].at[pl.ds(0, 16)],
                        cnt_out.at[cid * 32 + h * 16 + sid])


def _make_part():
    mesh = plsc.VectorSubcoreMesh(core_axis_name="c", subcore_axis_name="s")
    out_type = [
        jax.ShapeDtypeStruct((2 * REG,), jnp.int32),
        jax.ShapeDtypeStruct((2 * REG,), jnp.int32),
        jax.ShapeDtypeStruct((2 * REG,), jnp.int32),
        jax.ShapeDtypeStruct((2 * REG,), jnp.float32),
        jax.ShapeDtypeStruct((64, 16), jnp.int32),
    ]
    scratch = [pltpu.VMEM((CE,), jnp.int32)] * 3 + [
        pltpu.VMEM((CE,), jnp.float32)]
    for _ in range(2):
        scratch += [pltpu.VMEM((STG,), jnp.int32)] * 3 + [
            pltpu.VMEM((STG,), jnp.float32)]
    return functools.partial(
        pl.kernel, mesh=mesh, out_type=out_type, scratch_types=scratch,
        compiler_params=_SC_PARAMS,
    )(_part_body)


HALVES = ((0, 208), (208, 192))


def _agg_body(with_w, *refs):
    if with_w:
        (dst_hbm, idx_hbm, w_hbm, cnt_hbm, tbl_hbm, acc_out,
         ed_dst, ed_idx, ed_w, cntv,
         dstloc0, dstloc1, payload, sem, sem_s0, sem_s1, acc_sh) = refs
    else:
        (dst_hbm, idx_hbm, cnt_hbm, tbl_hbm, acc_out, ed_dst, ed_idx, cntv,
         dstloc0, dstloc1, payload, sem, sem_s0, sem_s1, acc_sh) = refs
    dstlocs = (dstloc0, dstloc1)
    ssems = (sem_s0, sem_s1)

    cid = lax.axis_index("c")
    sid = lax.axis_index("s")
    ii = lax.iota(jnp.int32, 16)
    z16 = jnp.zeros((16,), jnp.float32)

    def zp(i, carry):
        r16 = jnp.full((16,), i, jnp.int32)
        for s in range(4):
            plsc.store_scatter(payload, [r16, ii + 16 * s], z16)
        return carry
    lax.fori_loop(0, CE, zp, 0)

    row0 = sid * TROWS
    for k in range(4):
        pltpu.sync_copy(payload.at[pl.ds(0, 392)],
                        acc_sh.at[pl.ds(row0 + 392 * k, 392)])
    plsc.subcore_barrier()

    ebase = (cid * 16 + sid) * REGC
    lo = cid * HALF
    pltpu.sync_copy(cnt_hbm.at[cid * 16 + sid], cntv)
    nchunks = jnp.max(cntv[...])

    def load_chunk(c, bb):
        e0 = ebase + c * CE
        pltpu.sync_copy(dst_hbm.at[pl.ds(e0, CE)], ed_dst)
        pltpu.sync_copy(idx_hbm.at[pl.ds(e0, CE)], ed_idx.at[bb])
        if with_w:
            pltpu.sync_copy(w_hbm.at[pl.ds(e0, CE)], ed_w.at[bb])
        for v in range(CE // 16):
            d = ed_dst[pl.ds(16 * v, 16)]
            local = d - lo
            inr = (local >= 0) & (local < HALF)
            padr = PAD0 + ((ii + 16 * v) & 63)
            loc2 = jnp.where(inr, local, padr)
            if 16 * v < HALVES[0][1]:
                dstloc0[bb, pl.ds(16 * v, 16)] = loc2
            else:
                dstloc1[bb, pl.ds(16 * v - HALVES[0][1], 16)] = loc2

    def gather(bb, h):
        off, ln = HALVES[h]
        return pltpu.make_async_copy(
            tbl_hbm.at[ed_idx.at[bb, pl.ds(off, ln)]],
            payload.at[pl.ds(off, ln)], sem)

    def issue_gather(bb, h):
        off, ln = HALVES[h]
        pltpu.async_copy(
            tbl_hbm.at[ed_idx.at[bb, pl.ds(off, ln)]],
            payload.at[pl.ds(off, ln)], sem)

    def wmul_half(bb, h):
        off, ln = HALVES[h]

        def wmul(g, c2):
            for u in range(8):
                j = g * 8 + u
                wv = plsc.load_gather(
                    ed_w, [jnp.full((16,), bb, jnp.int32),
                           jnp.full((16,), off + j, jnp.int32)])
                for s in range(4):
                    sl = (off + j, pl.ds(16 * s, 16))
                    payload[sl] = payload[sl] * wv
            return c2
        lax.fori_loop(0, ln // 8, wmul, 0)

    def scatter_half(bb, h):
        off, ln = HALVES[h]
        pltpu.async_copy(payload.at[pl.ds(off, ln)],
                         acc_sh.at[dstlocs[h].at[bb]], ssems[h], add=True)

    def wait_scatter(bb, h):
        off, ln = HALVES[h]
        pltpu.make_async_copy(payload.at[pl.ds(off, ln)],
                              acc_sh.at[dstlocs[h].at[bb]], ssems[h]).wait()

    load_chunk(0, 0)
    issue_gather(0, 0)

    def chunk(c, carry):
        bb = c & 1
        nbb = 1 - bb

        @pl.when(c > 0)
        def _():
            wait_scatter(nbb, 1)

        @pl.when(c + 1 < nchunks)
        def _():
            load_chunk(c + 1, nbb)
        gather(bb, 0).wait()
        issue_gather(bb, 1)
        if with_w:
            wmul_half(bb, 0)
        scatter_half(bb, 0)
        gather(bb, 1).wait()
        wait_scatter(bb, 0)

        @pl.when(c + 1 < nchunks)
        def _():
            issue_gather(nbb, 0)
        if with_w:
            wmul_half(bb, 1)
        scatter_half(bb, 1)
        return carry
    lax.fori_loop(0, nchunks, chunk, 0)
    wait_scatter((nchunks - 1) & 1, 1)
    plsc.subcore_barrier()

    pltpu.sync_copy(acc_sh.at[pl.ds(row0, TROWS)],
                    acc_out.at[pl.ds(cid * ACC_ROWS + row0, TROWS)])


_SC_PARAMS = pltpu.CompilerParams(
    needs_layout_passes=False, use_tc_tiling_on_sc=False)


def _make_agg(with_w):
    mesh = plsc.VectorSubcoreMesh(core_axis_name="c", subcore_axis_name="s")
    out_type = [jax.ShapeDtypeStruct((2 * ACC_ROWS, H0), jnp.float32)]
    scratch = [
        pltpu.VMEM((CE,), jnp.int32),
        pltpu.VMEM((2, CE), jnp.int32),
    ]
    if with_w:
        scratch.append(pltpu.VMEM((2, CE), jnp.float32))
    scratch.append(pltpu.VMEM((16,), jnp.int32))
    scratch += [
        pltpu.VMEM((2, HALVES[0][1]), jnp.int32),
        pltpu.VMEM((2, HALVES[1][1]), jnp.int32),
        pltpu.VMEM((CE, H0), jnp.float32),
        pltpu.SemaphoreType.DMA,
        pltpu.SemaphoreType.DMA,
        pltpu.SemaphoreType.DMA,
    ]
    scratch.append(pltpu.VMEM_SHARED((ACC_ROWS, H0), jnp.float32))
    return functools.partial(
        pl.kernel, mesh=mesh, out_type=out_type, scratch_types=scratch,
        compiler_params=_SC_PARAMS,
    )(functools.partial(_agg_body, with_w))


DEG_ROWS = 50176


def _deg_body(dst2_hbm, deg_out, ed_dst, degloc):
    cid = lax.axis_index("c")
    sid = lax.axis_index("s")
    ii = lax.iota(jnp.int32, 16)
    z16 = jnp.zeros((16,), jnp.float32)
    ones16 = jnp.ones((16,), jnp.float32)

    def zd(i, carry):
        plsc.store_scatter(degloc, [i * 16 + ii], z16)
        return carry
    lax.fori_loop(0, DEG_ROWS // 16, zd, 0)

    ebase = cid * E + sid * EPT

    def chunk(i, carry):
        pltpu.sync_copy(dst2_hbm.at[pl.ds(ebase + i * CE, CE)], ed_dst)
        for v in range(CE // 16):
            d = ed_dst[pl.ds(16 * v, 16)]
            plsc.addupdate_scatter(degloc, [d], ones16)
        return carry
    lax.fori_loop(0, NCHUNK, chunk, 0)
    pltpu.sync_copy(degloc, deg_out.at[cid * 16 + sid])


def _make_deg():
    mesh = plsc.VectorSubcoreMesh(core_axis_name="c", subcore_axis_name="s")
    return functools.partial(
        pl.kernel, mesh=mesh,
        out_type=[jax.ShapeDtypeStruct((32, DEG_ROWS), jnp.float32)],
        scratch_types=[
            pltpu.VMEM((CE,), jnp.int32),
            pltpu.VMEM((DEG_ROWS,), jnp.float32),
        ],
        compiler_params=_SC_PARAMS,
    )(_deg_body)


def _combine_body(accg_ref, accc_ref, deg_ref, ctx_ref, wf_ref, bf_ref,
                  w1_ref, w2_ref, out_ref):
    dg = jnp.maximum(jnp.sum(deg_ref[...], axis=1), 1.0)
    recip = (1.0 / dg)[:, None]
    g = jax.nn.relu(accg_ref[...] * recip)
    c = jax.nn.relu(accc_ref[...] * recip)
    h = jax.nn.relu(
        jnp.dot(ctx_ref[...], wf_ref[...],
                preferred_element_type=jnp.float32) + bf_ref[...])
    w1 = w1_ref[...]
    w2 = w2_ref[...]
    out = jnp.dot(g, w1[:H0] + w2[:H0], preferred_element_type=jnp.float32)
    out += jnp.dot(h, w1[H0:], preferred_element_type=jnp.float32)
    out += jnp.dot(c, w2[H0:], preferred_element_type=jnp.float32)
    out_ref[...] = out


def _combine(accg, accc, deg, ctx, wf, bf, w1, w2):
    nb = 1000
    nblk = NU // nb
    hb = HALF // nb
    return pl.pallas_call(
        _combine_body,
        grid=(nblk,),
        in_specs=[
            pl.BlockSpec((nb, H0), lambda i: (i, 0)),
            pl.BlockSpec((nb, H0), lambda i: (i, 0)),
            pl.BlockSpec((nb, 16), lambda i: (i, 0)),
            pl.BlockSpec((nb, CF), lambda i: (i, 0)),
            pl.BlockSpec((CF, FH), lambda i: (0, 0)),
            pl.BlockSpec((1, FH), lambda i: (0, 0)),
            pl.BlockSpec((H0 + FH, H2), lambda i: (0, 0)),
            pl.BlockSpec((2 * H0, H2), lambda i: (0, 0)),
        ],
        out_specs=pl.BlockSpec((nb, H2), lambda i: (i, 0)),
        out_shape=jax.ShapeDtypeStruct((NU, H2), jnp.float32),
    )(accg, accc, deg, ctx, wf, bf, w1, w2)


def _halves(acc):
    return jnp.concatenate([acc[:HALF], acc[ACC_ROWS:ACC_ROWS + HALF]], axis=0)


def kernel(u_features, v_features, edge_index, edge_type, edge_ctx_weight,
           u_context, v_context, W_gcn, W_cgcn, W_fu, b_fu, W_fv, b_fv,
           W1_u, W1_v, W2_u, W2_v):
    row = edge_index[0].astype(jnp.int32)
    col = edge_index[1].astype(jnp.int32)
    et = edge_type.astype(jnp.int32)
    w = edge_ctx_weight.astype(jnp.float32)

    feats = jnp.concatenate([u_features, v_features], axis=0)
    t_tbl, c_tbl = _make_tables(feats, W_gcn, W_cgcn)
    t_flat = t_tbl.reshape(R * N, H0)

    e2 = (E // 128, 128)
    gu2, gv2, cu2 = _edge_indices(row.reshape(e2), col.reshape(e2),
                                  et.reshape(e2))
    gu = gu2.reshape(E)
    gv = gv2.reshape(E)
    cu = cu2.reshape(E)

    dst2 = jnp.concatenate([row, col])
    dstp, gidxp, cidxp, wp, cnt = _make_part()(
        dst2, jnp.concatenate([gu, gv]), jnp.concatenate([cu, row]), w)

    agg_gcn = _make_agg(False)
    agg_ctx = _make_agg(True)

    (accu_g,) = agg_gcn(dstp[:REG], gidxp[:REG], cnt[:32], t_flat)
    (accu_c,) = agg_ctx(dstp[:REG], cidxp[:REG], wp[:REG], cnt[:32], c_tbl)
    (accv_g,) = agg_gcn(dstp[REG:], gidxp[REG:], cnt[32:], t_flat)
    (accv_c,) = agg_ctx(dstp[REG:], cidxp[REG:], wp[REG:], cnt[32:], c_tbl)
    (deg2,) = _make_deg()(dst2)
    degu = deg2[:16, :NU].T
    degv = deg2[16:, :NU].T

    bfu = b_fu.reshape(1, FH)
    bfv = b_fv.reshape(1, FH)
    out_u = _combine(_halves(accu_g), _halves(accu_c), degu,
                     u_context, W_fu, bfu, W1_u, W2_u)
    out_v = _combine(_halves(accv_g), _halves(accv_c), degv,
                     v_context, W_fv, bfv, W1_v, W2_v)
    return jnp.concatenate([out_u, out_v], axis=0)


# speedup vs baseline: 1.2064x; 1.0126x over previous
import functools

import jax
import jax.numpy as jnp
from jax import lax
from jax.experimental import pallas as pl
from jax.experimental.pallas import tpu as pltpu
from jax.experimental.pallas import tpu_sc as plsc

NU = 50000
NI = 50000
N = NU + NI
E = 800000
D = 64
H0 = 64
H2 = 32
R = 5
CF = 16
FH = 64

HALF = 25000
ACC_ROWS = 25088
PAD0 = HALF
CE = 400
EPT = E // 16
NCHUNK = EPT // CE
TROWS = ACC_ROWS // 16


def _tables_body(feats_ref, wg_ref, wc_ref, t_ref, c_ref):
    w = wg_ref[...]
    blocks = [w[0]]
    for r in range(1, R):
        blocks.append(blocks[-1] + w[r])
    blocks.append(wc_ref[...])
    wst = jnp.concatenate(blocks, axis=1)
    y = jnp.dot(feats_ref[...], wst, preferred_element_type=jnp.float32)
    for r in range(R):
        t_ref[r] = y[:, r * H0:(r + 1) * H0]
    c_ref[...] = y[:, R * H0:]


def _make_tables(feats, w_gcn, w_cgcn):
    nb = 2000
    grid = (N // nb,)
    return pl.pallas_call(
        _tables_body,
        grid=grid,
        in_specs=[
            pl.BlockSpec((nb, D), lambda i: (i, 0)),
            pl.BlockSpec((R, D, H0), lambda i: (0, 0, 0)),
            pl.BlockSpec((D, H0), lambda i: (0, 0)),
        ],
        out_specs=[
            pl.BlockSpec((R, nb, H0), lambda i: (0, i, 0)),
            pl.BlockSpec((nb, H0), lambda i: (i, 0)),
        ],
        out_shape=[
            jax.ShapeDtypeStruct((R, N, H0), jnp.float32),
            jax.ShapeDtypeStruct((N, H0), jnp.float32),
        ],
    )(feats, w_gcn, w_cgcn)


def _eidx_body(row_ref, col_ref, et_ref, gu_ref, gv_ref, cu_ref):
    row = row_ref[...]
    col = col_ref[...]
    et = et_ref[...]
    gu_ref[...] = et * N + (col + NU)
    gv_ref[...] = et * N + row
    cu_ref[...] = col + NU


def _edge_indices(row2d, col2d, et2d):
    grid = (1,)
    spec = pl.BlockSpec(row2d.shape, lambda i: (0, 0))
    return pl.pallas_call(
        _eidx_body,
        grid=grid,
        in_specs=[spec, spec, spec],
        out_specs=[spec, spec, spec],
        out_shape=[jax.ShapeDtypeStruct(row2d.shape, jnp.int32)] * 3,
    )(row2d, col2d, et2d)


REGC = EPT + CE
REG = 32 * REGC
STG = CE + 16


def _part_body(dst2_hbm, gidx2_hbm, cidx2_hbm, w_hbm,
               dstp, gidxp, cidxp, wp, cnt_out,
               b_dst, b_gidx, b_cidx, b_w, *stg):
    cid = lax.axis_index("c")
    sid = lax.axis_index("s")
    ii = lax.iota(jnp.int32, 16)
    S = (stg[0:4], stg[4:8])
    outs = (dstp, gidxp, cidxp, wp)

    def flush(h, nf):
        base = (cid * 32 + h * 16 + sid) * REGC + nf * CE
        for a in range(4):
            pltpu.sync_copy(S[h][a].at[pl.ds(0, CE)],
                            outs[a].at[pl.ds(base, CE)])
        for a in range(4):
            t16 = S[h][a][pl.ds(CE, 16)]
            S[h][a][pl.ds(0, 16)] = t16

    def chunk(i, carry):
        off0, nf0, off1, nf1 = carry
        e0 = cid * E + sid * EPT + i * CE
        ew = sid * EPT + i * CE
        pltpu.sync_copy(dst2_hbm.at[pl.ds(e0, CE)], b_dst)
        pltpu.sync_copy(gidx2_hbm.at[pl.ds(e0, CE)], b_gidx)
        pltpu.sync_copy(cidx2_hbm.at[pl.ds(e0, CE)], b_cidx)
        pltpu.sync_copy(w_hbm.at[pl.ds(ew, CE)], b_w)
        for v in range(CE // 16):
            sl = pl.ds(16 * v, 16)
            d = b_dst[sl]
            vals = (d, b_gidx[sl], b_cidx[sl], b_w[sl])
            m0 = d < HALF
            offs = [off0, off1]
            nfs = [nf0, nf1]
            for h in range(2):
                m = m0 if h == 0 else jnp.logical_not(m0)
                off = offs[h]
                for a in range(4):
                    plsc.store_compressed(S[h][a].at[pl.ds(off, 16)],
                                          vals[a], mask=m)
                off = off + plsc.all_reduce_population_count(m)[0]
                do = off >= CE

                @pl.when(do)
                def _(h=h, nf=nfs[h]):
                    flush(h, nf)
                offs[h] = jnp.where(do, off - CE, off)
                nfs[h] = nfs[h] + do.astype(jnp.int32)
            off0, off1 = offs
            nf0, nf1 = nfs
        return off0, nf0, off1, nf1
    off0, nf0, off1, nf1 = lax.fori_loop(
        0, NCHUNK, chunk, (jnp.int32(0), jnp.int32(0),
                           jnp.int32(0), jnp.int32(0)))

    dpad = jnp.full((16,), -1, jnp.int32)
    spread = ii & 63
    wpad = jnp.zeros((16,), jnp.float32)
    pads = (dpad, spread, spread, wpad)
    for h, off, nf in ((0, off0, nf0), (1, off1, nf1)):
        for k in range(CE // 16):
            @pl.when(off + 16 * k < CE)
            def _(h=h, k=k, off=off):
                for a in range(4):
                    S[h][a][pl.ds(off + 16 * k, 16)] = pads[a]
        flush(h, nf)
        S[0][0][pl.ds(0, 16)] = jnp.full((16,), nf + 1, jnp.int32)
        pltpu.sync_copy(S[0][0].at[pl.ds(0, 16)],
                        cnt_out.at[cid * 32 + h * 16 + sid])


def _make_part():
    mesh = plsc.VectorSubcoreMesh(core_axis_name="c", subcore_axis_name="s")
    out_type = [
        jax.ShapeDtypeStruct((2 * REG,), jnp.int32),
        jax.ShapeDtypeStruct((2 * REG,), jnp.int32),
        jax.ShapeDtypeStruct((2 * REG,), jnp.int32),
        jax.ShapeDtypeStruct((2 * REG,), jnp.float32),
        jax.ShapeDtypeStruct((64, 16), jnp.int32),
    ]
    scratch = [pltpu.VMEM((CE,), jnp.int32)] * 3 + [
        pltpu.VMEM((CE,), jnp.float32)]
    for _ in range(2):
        scratch += [pltpu.VMEM((STG,), jnp.int32)] * 3 + [
            pltpu.VMEM((STG,), jnp.float32)]
    return functools.partial(
        pl.kernel, mesh=mesh, out_type=out_type, scratch_types=scratch,
        compiler_params=_SC_PARAMS,
    )(_part_body)


HALVES = ((0, 208), (208, 192))


def _agg_body(with_w, *refs):
    if with_w:
        (dst_hbm, idx_hbm, w_hbm, cnt_hbm, tbl_hbm, acc_out,
         ed_dst, ed_idx, ed_w, cntv,
         dstloc0, dstloc1, payload, sem, sem_s0, sem_s1, acc_sh) = refs
    else:
        (dst_hbm, idx_hbm, cnt_hbm, tbl_hbm, acc_out, ed_dst, ed_idx, cntv,
         dstloc0, dstloc1, payload, sem, sem_s0, sem_s1, acc_sh) = refs
    dstlocs = (dstloc0, dstloc1)
    ssems = (sem_s0, sem_s1)

    cid = lax.axis_index("c")
    sid = lax.axis_index("s")
    ii = lax.iota(jnp.int32, 16)
    z16 = jnp.zeros((16,), jnp.float32)

    def zp(i, carry):
        r16 = jnp.full((16,), i, jnp.int32)
        for s in range(4):
            plsc.store_scatter(payload, [r16, ii + 16 * s], z16)
        return carry
    lax.fori_loop(0, CE, zp, 0)

    row0 = sid * TROWS
    for k in range(4):
        pltpu.sync_copy(payload.at[pl.ds(0, 392)],
                        acc_sh.at[pl.ds(row0 + 392 * k, 392)])
    plsc.subcore_barrier()

    ebase = (cid * 16 + sid) * REGC
    lo = cid * HALF
    pltpu.sync_copy(cnt_hbm.at[cid * 16 + sid], cntv)
    nchunks = jnp.max(cntv[...])

    def load_chunk(c, bb):
        e0 = ebase + c * CE
        pltpu.sync_copy(dst_hbm.at[pl.ds(e0, CE)], ed_dst)
        pltpu.sync_copy(idx_hbm.at[pl.ds(e0, CE)], ed_idx.at[bb])
        if with_w:
            pltpu.sync_copy(w_hbm.at[pl.ds(e0, CE)], ed_w.at[bb])
        for v in range(CE // 16):
            d = ed_dst[pl.ds(16 * v, 16)]
            local = d - lo
            inr = (local >= 0) & (local < HALF)
            padr = PAD0 + ((ii + 16 * v) & 63)
            loc2 = jnp.where(inr, local, padr)
            if 16 * v < HALVES[0][1]:
                dstloc0[bb, pl.ds(16 * v, 16)] = loc2
            else:
                dstloc1[bb, pl.ds(16 * v - HALVES[0][1], 16)] = loc2

    def gather(bb, h):
        off, ln = HALVES[h]
        return pltpu.make_async_copy(
            tbl_hbm.at[ed_idx.at[bb, pl.ds(off, ln)]],
            payload.at[pl.ds(off, ln)], sem)

    def issue_gather(bb, h):
        off, ln = HALVES[h]
        pltpu.async_copy(
            tbl_hbm.at[ed_idx.at[bb, pl.ds(off, ln)]],
            payload.at[pl.ds(off, ln)], sem)

    def wmul_half(bb, h):
        off, ln = HALVES[h]

        def wmul(g, c2):
            for u in range(8):
                j = g * 8 + u
                wv = plsc.load_gather(
                    ed_w, [jnp.full((16,), bb, jnp.int32),
                           jnp.full((16,), off + j, jnp.int32)])
                for s in range(4):
                    sl = (off + j, pl.ds(16 * s, 16))
                    payload[sl] = payload[sl] * wv
            return c2
        lax.fori_loop(0, ln // 8, wmul, 0)

    def scatter_half(bb, h):
        off, ln = HALVES[h]
        pltpu.async_copy(payload.at[pl.ds(off, ln)],
                         acc_sh.at[dstlocs[h].at[bb]], ssems[h], add=True)

    def wait_scatter(bb, h):
        off, ln = HALVES[h]
        pltpu.make_async_copy(payload.at[pl.ds(off, ln)],
                              acc_sh.at[dstlocs[h].at[bb]], ssems[h]).wait()

    load_chunk(0, 0)
    issue_gather(0, 0)

    def chunk(c, carry):
        bb = c & 1
        nbb = 1 - bb

        @pl.when(c > 0)
        def _():
            wait_scatter(nbb, 1)

        @pl.when(c + 1 < nchunks)
        def _():
            load_chunk(c + 1, nbb)
        gather(bb, 0).wait()
        issue_gather(bb, 1)
        if with_w:
            wmul_half(bb, 0)
        scatter_half(bb, 0)
        gather(bb, 1).wait()
        wait_scatter(bb, 0)

        @pl.when(c + 1 < nchunks)
        def _():
            issue_gather(nbb, 0)
        if with_w:
            wmul_half(bb, 1)
        scatter_half(bb, 1)
        return carry
    lax.fori_loop(0, nchunks, chunk, 0)
    wait_scatter((nchunks - 1) & 1, 1)
    plsc.subcore_barrier()

    pltpu.sync_copy(acc_sh.at[pl.ds(row0, TROWS)],
                    acc_out.at[pl.ds(cid * ACC_ROWS + row0, TROWS)])


_SC_PARAMS = pltpu.CompilerParams(
    needs_layout_passes=False, use_tc_tiling_on_sc=False)


def _make_agg(with_w):
    mesh = plsc.VectorSubcoreMesh(core_axis_name="c", subcore_axis_name="s")
    out_type = [jax.ShapeDtypeStruct((2 * ACC_ROWS, H0), jnp.float32)]
    scratch = [
        pltpu.VMEM((CE,), jnp.int32),
        pltpu.VMEM((2, CE), jnp.int32),
    ]
    if with_w:
        scratch.append(pltpu.VMEM((2, CE), jnp.float32))
    scratch.append(pltpu.VMEM((16,), jnp.int32))
    scratch += [
        pltpu.VMEM((2, HALVES[0][1]), jnp.int32),
        pltpu.VMEM((2, HALVES[1][1]), jnp.int32),
        pltpu.VMEM((CE, H0), jnp.float32),
        pltpu.SemaphoreType.DMA,
        pltpu.SemaphoreType.DMA,
        pltpu.SemaphoreType.DMA,
    ]
    scratch.append(pltpu.VMEM_SHARED((ACC_ROWS, H0), jnp.float32))
    return functools.partial(
        pl.kernel, mesh=mesh, out_type=out_type, scratch_types=scratch,
        compiler_params=_SC_PARAMS,
    )(functools.partial(_agg_body, with_w))


DEG_ROWS = 50176


def _deg_body(dst2_hbm, deg_out, ed_dst, degloc):
    cid = lax.axis_index("c")
    sid = lax.axis_index("s")
    ii = lax.iota(jnp.int32, 16)
    z16 = jnp.zeros((16,), jnp.float32)
    ones16 = jnp.ones((16,), jnp.float32)

    def zd(i, carry):
        plsc.store_scatter(degloc, [i * 16 + ii], z16)
        return carry
    lax.fori_loop(0, DEG_ROWS // 16, zd, 0)

    ebase = cid * E + sid * EPT

    def chunk(i, carry):
        pltpu.sync_copy(dst2_hbm.at[pl.ds(ebase + i * CE, CE)], ed_dst)
        for v in range(CE // 16):
            d = ed_dst[pl.ds(16 * v, 16)]
            plsc.addupdate_scatter(degloc, [d], ones16)
        return carry
    lax.fori_loop(0, NCHUNK, chunk, 0)
    pltpu.sync_copy(degloc, deg_out.at[cid * 16 + sid])


def _make_deg():
    mesh = plsc.VectorSubcoreMesh(core_axis_name="c", subcore_axis_name="s")
    return functools.partial(
        pl.kernel, mesh=mesh,
        out_type=[jax.ShapeDtypeStruct((32, DEG_ROWS), jnp.float32)],
        scratch_types=[
            pltpu.VMEM((CE,), jnp.int32),
            pltpu.VMEM((DEG_ROWS,), jnp.float32),
        ],
        compiler_params=_SC_PARAMS,
    )(_deg_body)


def _combine_body(accg_ref, accc_ref, deg_ref, ctx_ref, wf_ref, bf_ref,
                  w1_ref, w2_ref, out_ref):
    dg = jnp.maximum(jnp.sum(deg_ref[...], axis=1), 1.0)
    recip = (1.0 / dg)[:, None]
    g = jax.nn.relu(accg_ref[...] * recip)
    c = jax.nn.relu(accc_ref[...] * recip)
    h = jax.nn.relu(
        jnp.dot(ctx_ref[...], wf_ref[...],
                preferred_element_type=jnp.float32) + bf_ref[...])
    w1 = w1_ref[...]
    w2 = w2_ref[...]
    out = jnp.dot(g, w1[:H0] + w2[:H0], preferred_element_type=jnp.float32)
    out += jnp.dot(h, w1[H0:], preferred_element_type=jnp.float32)
    out += jnp.dot(c, w2[H0:], preferred_element_type=jnp.float32)
    out_ref[...] = out


def _combine(accg, accc, deg, ctx, wf, bf, w1, w2):
    nb = 1000
    nblk = NU // nb
    hb = HALF // nb
    return pl.pallas_call(
        _combine_body,
        grid=(nblk,),
        in_specs=[
            pl.BlockSpec((nb, H0), lambda i: (i, 0)),
            pl.BlockSpec((nb, H0), lambda i: (i, 0)),
            pl.BlockSpec((nb, 16), lambda i: (i, 0)),
            pl.BlockSpec((nb, CF), lambda i: (i, 0)),
            pl.BlockSpec((CF, FH), lambda i: (0, 0)),
            pl.BlockSpec((1, FH), lambda i: (0, 0)),
            pl.BlockSpec((H0 + FH, H2), lambda i: (0, 0)),
            pl.BlockSpec((2 * H0, H2), lambda i: (0, 0)),
        ],
        out_specs=pl.BlockSpec((nb, H2), lambda i: (i, 0)),
        out_shape=jax.ShapeDtypeStruct((NU, H2), jnp.float32),
    )(accg, accc, deg, ctx, wf, bf, w1, w2)


def _halves(acc):
    return jnp.concatenate([acc[:HALF], acc[ACC_ROWS:ACC_ROWS + HALF]], axis=0)


def kernel(u_features, v_features, edge_index, edge_type, edge_ctx_weight,
           u_context, v_context, W_gcn, W_cgcn, W_fu, b_fu, W_fv, b_fv,
           W1_u, W1_v, W2_u, W2_v):
    row = edge_index[0].astype(jnp.int32)
    col = edge_index[1].astype(jnp.int32)
    et = edge_type.astype(jnp.int32)
    w = edge_ctx_weight.astype(jnp.float32)

    feats = jnp.concatenate([u_features, v_features], axis=0)
    t_tbl, c_tbl = _make_tables(feats, W_gcn, W_cgcn)
    t_flat = t_tbl.reshape(R * N, H0)

    e2 = (E // 128, 128)
    gu2, gv2, cu2 = _edge_indices(row.reshape(e2), col.reshape(e2),
                                  et.reshape(e2))
    gu = gu2.reshape(E)
    gv = gv2.reshape(E)
    cu = cu2.reshape(E)

    dst2 = jnp.concatenate([row, col])
    dstp, gidxp, cidxp, wp, cnt = _make_part()(
        dst2, jnp.concatenate([gu, gv]), jnp.concatenate([cu, row]), w)

    agg_gcn = _make_agg(False)
    agg_ctx = _make_agg(True)

    (accu_g,) = agg_gcn(dstp[:REG], gidxp[:REG], cnt[:32], t_flat)
    (accu_c,) = agg_ctx(dstp[:REG], cidxp[:REG], wp[:REG], cnt[:32], c_tbl)
    (accv_g,) = agg_gcn(dstp[REG:], gidxp[REG:], cnt[32:], t_flat)
    (accv_c,) = agg_ctx(dstp[REG:], cidxp[REG:], wp[REG:], cnt[32:], c_tbl)
    (deg2,) = _make_deg()(dst2)
    degu = deg2[:16, :NU].T
    degv = deg2[16:, :NU].T

    bfu = b_fu.reshape(1, FH)
    bfv = b_fv.reshape(1, FH)
    out_u = _combine(_halves(accu_g), _halves(accu_c), degu,
                     u_context, W_fu, bfu, W1_u, W2_u)
    out_v = _combine(_halves(accv_g), _halves(accv_c), degv,
                     v_context, W_fv, bfv, W1_v, W2_v)
    return jnp.concatenate([out_u, out_v], axis=0)
